# Initial kernel scaffold; baseline (speedup 1.0000x reference)
import jax
import jax.numpy as jnp
from jax.experimental import pallas as pl


def kernel(x, edge_index, batch, W1, b1, W2, b2, W3, b3, Wc, bc):
    raise NotImplementedError("write your pallas kernel here")


# speedup vs baseline: 10.8341x; 10.8341x over previous
import functools

import jax
import jax.numpy as jnp
from jax import lax
from jax.experimental import pallas as pl
from jax.experimental.pallas import tpu as pltpu
from jax.experimental.pallas import tpu_sc as plsc

N = 10000
E = 320000
H = 128
C = 10
G = 64

NC, NS, L = 2, 16, 16
NW = NC * NS
K = 128
NCH = 80
EPT = NCH * K
EPAD = NW * EPT
NPAD = NS * 632
ZR = NPAD // NS
DW = 16

_mesh = plsc.VectorSubcoreMesh(
    core_axis_name="c", subcore_axis_name="s", num_cores=NC, num_subcores=NS)


@functools.partial(
    pl.kernel,
    out_type=jax.ShapeDtypeStruct((NC * NPAD, H), jnp.float32),
    mesh=_mesh,
    scratch_types=[
        pltpu.VMEM((K,), jnp.int32),
        pltpu.VMEM((K,), jnp.int32),
        pltpu.VMEM((K, H), jnp.float32),
        pltpu.VMEM_SHARED((NPAD, H), jnp.float32),
        pltpu.SemaphoreType.DMA,
    ],
)
def _msg_kernel(h_hbm, src_hbm, dst_hbm, out_hbm, sidx_v, didx_v, rows_v,
                acc_sh, sem):
    c = lax.axis_index("c")
    s = lax.axis_index("s")
    w = c * NS + s

    def _fill_zero(i, carry):
        for u in range(H // L):
            rows_v[i, pl.ds(u * L, L)] = jnp.zeros((L,), jnp.float32)
        return carry

    lax.fori_loop(0, K, _fill_zero, 0)
    base = s * ZR
    off = 0
    while off < ZR:
        n = min(K, ZR - off)
        pltpu.sync_copy(rows_v.at[pl.ds(0, n)],
                        acc_sh.at[pl.ds(base + off, n)])
        off += n
    plsc.subcore_barrier()

    def _chunk(j, carry):
        eoff = w * EPT + j * K
        pltpu.sync_copy(src_hbm.at[pl.ds(eoff, K)], sidx_v)
        pltpu.sync_copy(dst_hbm.at[pl.ds(eoff, K)], didx_v)
        pltpu.async_copy(h_hbm.at[sidx_v], rows_v, sem).wait()
        pltpu.sync_copy(rows_v, acc_sh.at[didx_v], add=True)
        return carry

    lax.fori_loop(0, NCH, _chunk, 0)
    plsc.subcore_barrier()
    pltpu.sync_copy(acc_sh.at[pl.ds(s * ZR, ZR)],
                    out_hbm.at[pl.ds(c * NPAD + s * ZR, ZR)])


R = 2000
NBLK = N // R
_PREC = lax.Precision.HIGHEST


def _dinv_of(d0_ref, d1_ref):
    return lax.rsqrt(d0_ref[:, 0:1] + d1_ref[:, 0:1] + 1.0)


def _tc1_body(x_ref, w_ref, d0_ref, d1_ref, o_ref):
    dinv = _dinv_of(d0_ref, d1_ref)
    o_ref[...] = dinv * jnp.dot(x_ref[...], w_ref[...],
                                preferred_element_type=jnp.float32,
                                precision=_PREC)


def _tc1(x, w, d0, d1):
    return pl.pallas_call(
        _tc1_body,
        grid=(NBLK,),
        in_specs=[
            pl.BlockSpec((R, H), lambda i: (i, 0)),
            pl.BlockSpec((H, H), lambda i: (0, 0)),
            pl.BlockSpec((R, DW), lambda i: (i, 0)),
            pl.BlockSpec((R, DW), lambda i: (i, 0)),
        ],
        out_specs=pl.BlockSpec((R, H), lambda i: (i, 0)),
        out_shape=jax.ShapeDtypeStruct((N, H), jnp.float32),
    )(x, w, d0, d1)


def _tc_mid_body(a0_ref, a1_ref, hp_ref, b_ref, w_ref, d0_ref, d1_ref, o_ref):
    dinv = _dinv_of(d0_ref, d1_ref)
    xn = jnp.maximum(
        dinv * (a0_ref[...] + a1_ref[...] + hp_ref[...]) + b_ref[...], 0.0)
    o_ref[...] = dinv * jnp.dot(xn, w_ref[...],
                                preferred_element_type=jnp.float32,
                                precision=_PREC)


def _tc_mid(a0, a1, hp, b, w, d0, d1):
    return pl.pallas_call(
        _tc_mid_body,
        grid=(NBLK,),
        in_specs=[
            pl.BlockSpec((R, H), lambda i: (i, 0)),
            pl.BlockSpec((R, H), lambda i: (i, 0)),
            pl.BlockSpec((R, H), lambda i: (i, 0)),
            pl.BlockSpec((1, H), lambda i: (0, 0)),
            pl.BlockSpec((H, H), lambda i: (0, 0)),
            pl.BlockSpec((R, DW), lambda i: (i, 0)),
            pl.BlockSpec((R, DW), lambda i: (i, 0)),
        ],
        out_specs=pl.BlockSpec((R, H), lambda i: (i, 0)),
        out_shape=jax.ShapeDtypeStruct((N, H), jnp.float32),
    )(a0, a1, hp, b, w, d0, d1)


def _tc_fin_body(a0_ref, a1_ref, hp_ref, b_ref, d0_ref, d1_ref, bt_ref,
                 wc_ref, bc_ref, o_ref, pool_s, cnt_s):
    i = pl.program_id(0)
    dinv = _dinv_of(d0_ref, d1_ref)
    x3 = jnp.maximum(
        dinv * (a0_ref[...] + a1_ref[...] + hp_ref[...]) + b_ref[...], 0.0)
    gid = lax.broadcasted_iota(jnp.int32, (R, G), 1)
    m = (gid == bt_ref[...]).astype(jnp.float32)
    dn = (((0,), (0,)), ((), ()))
    pm = lax.dot_general(m, x3, dimension_numbers=dn,
                         preferred_element_type=jnp.float32, precision=_PREC)
    cm = lax.dot_general(m, jnp.ones((R, H), jnp.float32),
                         dimension_numbers=dn,
                         preferred_element_type=jnp.float32, precision=_PREC)

    @pl.when(i == 0)
    def _():
        pool_s[...] = pm
        cnt_s[...] = cm

    @pl.when(i > 0)
    def _():
        pool_s[...] += pm
        cnt_s[...] += cm

    @pl.when(i == NBLK - 1)
    def _():
        pooled = pool_s[...] / jnp.maximum(cnt_s[...], 1.0)
        logits = jnp.dot(pooled, wc_ref[...],
                         preferred_element_type=jnp.float32,
                         precision=_PREC) + bc_ref[...]
        mx = jnp.max(logits, axis=1, keepdims=True)
        ez = jnp.exp(logits - mx)
        o_ref[...] = logits - mx - jnp.log(jnp.sum(ez, axis=1, keepdims=True))


def _tc_fin(a0, a1, hp, b, d0, d1, bt, wc, bc):
    return pl.pallas_call(
        _tc_fin_body,
        grid=(NBLK,),
        in_specs=[
            pl.BlockSpec((R, H), lambda i: (i, 0)),
            pl.BlockSpec((R, H), lambda i: (i, 0)),
            pl.BlockSpec((R, H), lambda i: (i, 0)),
            pl.BlockSpec((1, H), lambda i: (0, 0)),
            pl.BlockSpec((R, DW), lambda i: (i, 0)),
            pl.BlockSpec((R, DW), lambda i: (i, 0)),
            pl.BlockSpec((R, 1), lambda i: (i, 0)),
            pl.BlockSpec((H, C), lambda i: (0, 0)),
            pl.BlockSpec((1, C), lambda i: (0, 0)),
        ],
        out_specs=pl.BlockSpec((G, C), lambda i: (0, 0)),
        out_shape=jax.ShapeDtypeStruct((G, C), jnp.float32),
        scratch_shapes=[
            pltpu.VMEM((G, H), jnp.float32),
            pltpu.VMEM((G, H), jnp.float32),
        ],
    )(a0, a1, hp, b, d0, d1, bt, wc, bc)


def _split(m):
    return m[:NPAD], m[NPAD:]


def kernel(x, edge_index, batch, W1, b1, W2, b2, W3, b3, Wc, bc):
    pad = EPAD - E
    cyc = jnp.arange(pad, dtype=jnp.int32) % (NPAD - N)
    srcp = jnp.concatenate([edge_index[0], cyc])
    dstp = jnp.concatenate([edge_index[1], N + cyc])

    ones = jnp.ones((N, H), jnp.float32)
    mdeg = _msg_kernel(ones, srcp, dstp)
    d0 = mdeg[:NPAD, :DW]
    d1 = mdeg[NPAD:, :DW]

    h1p = _tc1(x, W1, d0, d1)
    m1a, m1b = _split(_msg_kernel(h1p, srcp, dstp))
    h2p = _tc_mid(m1a, m1b, h1p, b1.reshape(1, H), W2, d0, d1)
    m2a, m2b = _split(_msg_kernel(h2p, srcp, dstp))
    h3p = _tc_mid(m2a, m2b, h2p, b2.reshape(1, H), W3, d0, d1)
    m3a, m3b = _split(_msg_kernel(h3p, srcp, dstp))
    return _tc_fin(m3a, m3b, h3p, b3.reshape(1, H), d0, d1,
                   batch.reshape(N, 1), Wc, bc.reshape(1, C))


# speedup vs baseline: 18.5733x; 1.7143x over previous
import functools

import jax
import jax.numpy as jnp
from jax import lax
from jax.experimental import pallas as pl
from jax.experimental.pallas import tpu as pltpu
from jax.experimental.pallas import tpu_sc as plsc

N = 10000
E = 320000
H = 128
C = 10
G = 64

NC, NS, L = 2, 16, 16
NW = NC * NS
K = 128
NCH = 80
EPT = NCH * K
EPAD = NW * EPT
NPAD = NS * 632
ZR = NPAD // NS
DW = 16

_mesh = plsc.VectorSubcoreMesh(
    core_axis_name="c", subcore_axis_name="s", num_cores=NC, num_subcores=NS)


@functools.partial(
    pl.kernel,
    out_type=jax.ShapeDtypeStruct((NC * NPAD, H), jnp.float32),
    mesh=_mesh,
    scratch_types=[
        pltpu.VMEM((EPT // 2,), jnp.int32),
        pltpu.VMEM((NCH // 2, K), jnp.int32),
        pltpu.VMEM((K, H), jnp.float32),
        pltpu.VMEM((K, H), jnp.float32),
        pltpu.VMEM_SHARED((NPAD, H), jnp.float32),
        pltpu.SemaphoreType.DMA,
        pltpu.SemaphoreType.DMA,
    ],
)
def _msg_kernel(h_hbm, src_hbm, dst_hbm, out_hbm, sidx_v, didx_v, rows0,
                rows1, acc_sh, sem0, sem1):
    c = lax.axis_index("c")
    s = lax.axis_index("s")
    w = c * NS + s

    def _fill_zero(i, carry):
        for u in range(H // L):
            rows0[i, pl.ds(u * L, L)] = jnp.zeros((L,), jnp.float32)
        return carry

    lax.fori_loop(0, K, _fill_zero, 0)
    base = s * ZR
    off = 0
    while off < ZR:
        n = min(K, ZR - off)
        pltpu.sync_copy(rows0.at[pl.ds(0, n)],
                        acc_sh.at[pl.ds(base + off, n)])
        off += n

    plsc.subcore_barrier()

    NCH2 = NCH // 2
    for half in range(2):
        pltpu.sync_copy(
            src_hbm.at[pl.ds(w * EPT + half * (EPT // 2), EPT // 2)], sidx_v)
        pltpu.sync_copy(dst_hbm.at[w, pl.ds(half * NCH2, NCH2)], didx_v)
        pltpu.async_copy(h_hbm.at[sidx_v.at[pl.ds(0, K)]], rows0, sem0)

        def _pair(i, carry):
            j0 = 2 * i
            pltpu.make_async_copy(
                h_hbm.at[sidx_v.at[pl.ds(j0 * K, K)]], rows0, sem0).wait()
            pltpu.async_copy(
                h_hbm.at[sidx_v.at[pl.ds(j0 * K + K, K)]], rows1, sem1)
            pltpu.sync_copy(rows0, acc_sh.at[didx_v.at[j0]], add=True)
            pltpu.make_async_copy(
                h_hbm.at[sidx_v.at[pl.ds(j0 * K + K, K)]], rows1, sem1).wait()
            nxt = jnp.minimum(j0 + 2, NCH2 - 1) * K
            pltpu.async_copy(h_hbm.at[sidx_v.at[pl.ds(nxt, K)]], rows0, sem0)
            pltpu.sync_copy(rows1, acc_sh.at[didx_v.at[j0 + 1]], add=True)
            return carry

        lax.fori_loop(0, NCH2 // 2, _pair, 0)
        pltpu.make_async_copy(
            h_hbm.at[sidx_v.at[pl.ds(0, K)]], rows0, sem0).wait()
    plsc.subcore_barrier()
    pltpu.sync_copy(acc_sh.at[pl.ds(s * ZR, ZR)],
                    out_hbm.at[pl.ds(c * NPAD + s * ZR, ZR)])


R = 2000
NBLK = N // R
_PREC = lax.Precision.HIGHEST


def _dinv_of(d0_ref, d1_ref):
    return lax.rsqrt(d0_ref[:, 0:1] + d1_ref[:, 0:1] + 1.0)


def _tc1_body(x_ref, w_ref, d0_ref, d1_ref, o_ref):
    dinv = _dinv_of(d0_ref, d1_ref)
    o_ref[...] = dinv * jnp.dot(x_ref[...], w_ref[...],
                                preferred_element_type=jnp.float32,
                                precision=_PREC)


def _tc1(x, w, d0, d1):
    return pl.pallas_call(
        _tc1_body,
        grid=(NBLK,),
        in_specs=[
            pl.BlockSpec((R, H), lambda i: (i, 0)),
            pl.BlockSpec((H, H), lambda i: (0, 0)),
            pl.BlockSpec((R, DW), lambda i: (i, 0)),
            pl.BlockSpec((R, DW), lambda i: (i, 0)),
        ],
        out_specs=pl.BlockSpec((R, H), lambda i: (i, 0)),
        out_shape=jax.ShapeDtypeStruct((N, H), jnp.float32),
    )(x, w, d0, d1)


def _tc_mid_body(a0_ref, a1_ref, hp_ref, b_ref, w_ref, d0_ref, d1_ref, o_ref):
    dinv = _dinv_of(d0_ref, d1_ref)
    xn = jnp.maximum(
        dinv * (a0_ref[...] + a1_ref[...] + hp_ref[...]) + b_ref[...], 0.0)
    o_ref[...] = dinv * jnp.dot(xn, w_ref[...],
                                preferred_element_type=jnp.float32,
                                precision=_PREC)


def _tc_mid(a0, a1, hp, b, w, d0, d1):
    return pl.pallas_call(
        _tc_mid_body,
        grid=(NBLK,),
        in_specs=[
            pl.BlockSpec((R, H), lambda i: (i, 0)),
            pl.BlockSpec((R, H), lambda i: (i, 0)),
            pl.BlockSpec((R, H), lambda i: (i, 0)),
            pl.BlockSpec((1, H), lambda i: (0, 0)),
            pl.BlockSpec((H, H), lambda i: (0, 0)),
            pl.BlockSpec((R, DW), lambda i: (i, 0)),
            pl.BlockSpec((R, DW), lambda i: (i, 0)),
        ],
        out_specs=pl.BlockSpec((R, H), lambda i: (i, 0)),
        out_shape=jax.ShapeDtypeStruct((N, H), jnp.float32),
    )(a0, a1, hp, b, w, d0, d1)


def _tc_fin_body(a0_ref, a1_ref, hp_ref, b_ref, d0_ref, d1_ref, bt_ref,
                 wc_ref, bc_ref, o_ref, pool_s, cnt_s):
    i = pl.program_id(0)
    dinv = _dinv_of(d0_ref, d1_ref)
    x3 = jnp.maximum(
        dinv * (a0_ref[...] + a1_ref[...] + hp_ref[...]) + b_ref[...], 0.0)
    gid = lax.broadcasted_iota(jnp.int32, (R, G), 1)
    m = (gid == bt_ref[...]).astype(jnp.float32)
    dn = (((0,), (0,)), ((), ()))
    pm = lax.dot_general(m, x3, dimension_numbers=dn,
                         preferred_element_type=jnp.float32, precision=_PREC)
    cm = lax.dot_general(m, jnp.ones((R, H), jnp.float32),
                         dimension_numbers=dn,
                         preferred_element_type=jnp.float32, precision=_PREC)

    @pl.when(i == 0)
    def _():
        pool_s[...] = pm
        cnt_s[...] = cm

    @pl.when(i > 0)
    def _():
        pool_s[...] += pm
        cnt_s[...] += cm

    @pl.when(i == NBLK - 1)
    def _():
        pooled = pool_s[...] / jnp.maximum(cnt_s[...], 1.0)
        logits = jnp.dot(pooled, wc_ref[...],
                         preferred_element_type=jnp.float32,
                         precision=_PREC) + bc_ref[...]
        mx = jnp.max(logits, axis=1, keepdims=True)
        ez = jnp.exp(logits - mx)
        o_ref[...] = logits - mx - jnp.log(jnp.sum(ez, axis=1, keepdims=True))


def _tc_fin(a0, a1, hp, b, d0, d1, bt, wc, bc):
    return pl.pallas_call(
        _tc_fin_body,
        grid=(NBLK,),
        in_specs=[
            pl.BlockSpec((R, H), lambda i: (i, 0)),
            pl.BlockSpec((R, H), lambda i: (i, 0)),
            pl.BlockSpec((R, H), lambda i: (i, 0)),
            pl.BlockSpec((1, H), lambda i: (0, 0)),
            pl.BlockSpec((R, DW), lambda i: (i, 0)),
            pl.BlockSpec((R, DW), lambda i: (i, 0)),
            pl.BlockSpec((R, 1), lambda i: (i, 0)),
            pl.BlockSpec((H, C), lambda i: (0, 0)),
            pl.BlockSpec((1, C), lambda i: (0, 0)),
        ],
        out_specs=pl.BlockSpec((G, C), lambda i: (0, 0)),
        out_shape=jax.ShapeDtypeStruct((G, C), jnp.float32),
        scratch_shapes=[
            pltpu.VMEM((G, H), jnp.float32),
            pltpu.VMEM((G, H), jnp.float32),
        ],
    )(a0, a1, hp, b, d0, d1, bt, wc, bc)


def _split(m):
    return m[:NPAD], m[NPAD:]


def kernel(x, edge_index, batch, W1, b1, W2, b2, W3, b3, Wc, bc):
    pad = EPAD - E
    cyc = jnp.arange(pad, dtype=jnp.int32) % (NPAD - N)
    srcp = jnp.concatenate([edge_index[0], cyc])
    dstp = jnp.concatenate([edge_index[1], N + cyc]).reshape(NW, NCH, K)

    ones = jnp.ones((N, H), jnp.float32)
    mdeg = _msg_kernel(ones, srcp, dstp)
    d0 = mdeg[:NPAD, :DW]
    d1 = mdeg[NPAD:, :DW]

    h1p = _tc1(x, W1, d0, d1)
    m1a, m1b = _split(_msg_kernel(h1p, srcp, dstp))
    h2p = _tc_mid(m1a, m1b, h1p, b1.reshape(1, H), W2, d0, d1)
    m2a, m2b = _split(_msg_kernel(h2p, srcp, dstp))
    h3p = _tc_mid(m2a, m2b, h2p, b2.reshape(1, H), W3, d0, d1)
    m3a, m3b = _split(_msg_kernel(h3p, srcp, dstp))
    return _tc_fin(m3a, m3b, h3p, b3.reshape(1, H), d0, d1,
                   batch.reshape(N, 1), Wc, bc.reshape(1, C))


# speedup vs baseline: 20.5752x; 1.1078x over previous
import functools

import jax
import jax.numpy as jnp
from jax import lax
from jax.experimental import pallas as pl
from jax.experimental.pallas import tpu as pltpu
from jax.experimental.pallas import tpu_sc as plsc

N = 10000
E = 320000
H = 128
C = 10
G = 64

NC, NS, L = 2, 16, 16
NW = NC * NS
K = 128
NCH = 80
EPT = NCH * K
EPAD = NW * EPT
NPAD = NS * 632
ZR = NPAD // NS
DW = 16

_mesh = plsc.VectorSubcoreMesh(
    core_axis_name="c", subcore_axis_name="s", num_cores=NC, num_subcores=NS)


@functools.partial(
    pl.kernel,
    out_type=jax.ShapeDtypeStruct((NC * NPAD, H), jnp.float32),
    mesh=_mesh,
    scratch_types=[
        pltpu.VMEM((EPT // 2,), jnp.int32),
        pltpu.VMEM((NCH // 2, K), jnp.int32),
        pltpu.VMEM((K, H), jnp.float32),
        pltpu.VMEM((K, H), jnp.float32),
        pltpu.VMEM_SHARED((NPAD, H), jnp.float32),
        pltpu.SemaphoreType.DMA,
        pltpu.SemaphoreType.DMA,
    ],
)
def _msg_kernel(h_hbm, src_hbm, dst_hbm, out_hbm, sidx_v, didx_v, rows0,
                rows1, acc_sh, sem0, sem1):
    c = lax.axis_index("c")
    s = lax.axis_index("s")
    w = c * NS + s

    def _fill_zero(i, carry):
        for u in range(H // L):
            rows0[i, pl.ds(u * L, L)] = jnp.zeros((L,), jnp.float32)
        return carry

    lax.fori_loop(0, K, _fill_zero, 0)
    base = s * ZR
    off = 0
    while off < ZR:
        n = min(K, ZR - off)
        pltpu.sync_copy(rows0.at[pl.ds(0, n)],
                        acc_sh.at[pl.ds(base + off, n)])
        off += n

    plsc.subcore_barrier()

    NCH2 = NCH // 2
    for half in range(2):
        pltpu.sync_copy(
            src_hbm.at[pl.ds(w * EPT + half * (EPT // 2), EPT // 2)], sidx_v)
        pltpu.sync_copy(dst_hbm.at[w, pl.ds(half * NCH2, NCH2)], didx_v)
        pltpu.async_copy(h_hbm.at[sidx_v.at[pl.ds(0, K)]], rows0, sem0)

        def _pair(i, carry):
            j0 = 2 * i
            pltpu.make_async_copy(
                h_hbm.at[sidx_v.at[pl.ds(j0 * K, K)]], rows0, sem0).wait()
            pltpu.async_copy(
                h_hbm.at[sidx_v.at[pl.ds(j0 * K + K, K)]], rows1, sem1)
            pltpu.sync_copy(rows0, acc_sh.at[didx_v.at[j0]], add=True)
            pltpu.make_async_copy(
                h_hbm.at[sidx_v.at[pl.ds(j0 * K + K, K)]], rows1, sem1).wait()
            nxt = jnp.minimum(j0 + 2, NCH2 - 1) * K
            pltpu.async_copy(h_hbm.at[sidx_v.at[pl.ds(nxt, K)]], rows0, sem0)
            pltpu.sync_copy(rows1, acc_sh.at[didx_v.at[j0 + 1]], add=True)
            return carry

        lax.fori_loop(0, NCH2 // 2, _pair, 0)
        pltpu.make_async_copy(
            h_hbm.at[sidx_v.at[pl.ds(0, K)]], rows0, sem0).wait()
    plsc.subcore_barrier()
    pltpu.sync_copy(acc_sh.at[pl.ds(s * ZR, ZR)],
                    out_hbm.at[pl.ds(c * NPAD + s * ZR, ZR)])


@functools.partial(
    pl.kernel,
    out_type=jax.ShapeDtypeStruct((NC * NPAD, H), jnp.float32),
    mesh=_mesh,
    scratch_types=[
        pltpu.VMEM((NCH, K), jnp.int32),
        pltpu.VMEM((K, H), jnp.float32),
        pltpu.VMEM_SHARED((NPAD, H), jnp.float32),
    ],
)
def _deg_kernel(dst_hbm, out_hbm, didx_v, ones_v, acc_sh):
    c = lax.axis_index("c")
    s = lax.axis_index("s")
    w = c * NS + s

    def _fill(i, carry):
        for u in range(H // L):
            ones_v[i, pl.ds(u * L, L)] = jnp.zeros((L,), jnp.float32)
        return carry

    lax.fori_loop(0, K, _fill, 0)
    base = s * ZR
    off = 0
    while off < ZR:
        n = min(K, ZR - off)
        pltpu.sync_copy(ones_v.at[pl.ds(0, n)],
                        acc_sh.at[pl.ds(base + off, n)])
        off += n

    def _refill(i, carry):
        for u in range(H // L):
            ones_v[i, pl.ds(u * L, L)] = jnp.ones((L,), jnp.float32)
        return carry

    lax.fori_loop(0, K, _refill, 0)
    pltpu.sync_copy(dst_hbm.at[w], didx_v)
    plsc.subcore_barrier()

    def _chunk(j, carry):
        pltpu.sync_copy(ones_v, acc_sh.at[didx_v.at[j]], add=True)
        return carry

    lax.fori_loop(0, NCH, _chunk, 0)
    plsc.subcore_barrier()
    pltpu.sync_copy(acc_sh.at[pl.ds(s * ZR, ZR)],
                    out_hbm.at[pl.ds(c * NPAD + s * ZR, ZR)])


R = 2000
NBLK = N // R
_PREC = lax.Precision.HIGHEST


def _dinv_of(d0_ref, d1_ref):
    return lax.rsqrt(d0_ref[:, 0:1] + d1_ref[:, 0:1] + 1.0)


def _tc1_body(x_ref, w_ref, d0_ref, d1_ref, o_ref):
    dinv = _dinv_of(d0_ref, d1_ref)
    o_ref[...] = dinv * jnp.dot(x_ref[...], w_ref[...],
                                preferred_element_type=jnp.float32,
                                precision=_PREC)


def _tc1(x, w, d0, d1):
    return pl.pallas_call(
        _tc1_body,
        grid=(NBLK,),
        in_specs=[
            pl.BlockSpec((R, H), lambda i: (i, 0)),
            pl.BlockSpec((H, H), lambda i: (0, 0)),
            pl.BlockSpec((R, DW), lambda i: (i, 0)),
            pl.BlockSpec((R, DW), lambda i: (i, 0)),
        ],
        out_specs=pl.BlockSpec((R, H), lambda i: (i, 0)),
        out_shape=jax.ShapeDtypeStruct((N, H), jnp.float32),
    )(x, w, d0, d1)


def _tc_mid_body(a0_ref, a1_ref, hp_ref, b_ref, w_ref, d0_ref, d1_ref, o_ref):
    dinv = _dinv_of(d0_ref, d1_ref)
    xn = jnp.maximum(
        dinv * (a0_ref[...] + a1_ref[...] + hp_ref[...]) + b_ref[...], 0.0)
    o_ref[...] = dinv * jnp.dot(xn, w_ref[...],
                                preferred_element_type=jnp.float32,
                                precision=_PREC)


def _tc_mid(a0, a1, hp, b, w, d0, d1):
    return pl.pallas_call(
        _tc_mid_body,
        grid=(NBLK,),
        in_specs=[
            pl.BlockSpec((R, H), lambda i: (i, 0)),
            pl.BlockSpec((R, H), lambda i: (i, 0)),
            pl.BlockSpec((R, H), lambda i: (i, 0)),
            pl.BlockSpec((1, H), lambda i: (0, 0)),
            pl.BlockSpec((H, H), lambda i: (0, 0)),
            pl.BlockSpec((R, DW), lambda i: (i, 0)),
            pl.BlockSpec((R, DW), lambda i: (i, 0)),
        ],
        out_specs=pl.BlockSpec((R, H), lambda i: (i, 0)),
        out_shape=jax.ShapeDtypeStruct((N, H), jnp.float32),
    )(a0, a1, hp, b, w, d0, d1)


def _tc_fin_body(a0_ref, a1_ref, hp_ref, b_ref, d0_ref, d1_ref, bt_ref,
                 wc_ref, bc_ref, o_ref, pool_s, cnt_s):
    i = pl.program_id(0)
    dinv = _dinv_of(d0_ref, d1_ref)
    x3 = jnp.maximum(
        dinv * (a0_ref[...] + a1_ref[...] + hp_ref[...]) + b_ref[...], 0.0)
    gid = lax.broadcasted_iota(jnp.int32, (R, G), 1)
    m = (gid == bt_ref[...]).astype(jnp.float32)
    dn = (((0,), (0,)), ((), ()))
    pm = lax.dot_general(m, x3, dimension_numbers=dn,
                         preferred_element_type=jnp.float32, precision=_PREC)
    cm = lax.dot_general(m, jnp.ones((R, H), jnp.float32),
                         dimension_numbers=dn,
                         preferred_element_type=jnp.float32, precision=_PREC)

    @pl.when(i == 0)
    def _():
        pool_s[...] = pm
        cnt_s[...] = cm

    @pl.when(i > 0)
    def _():
        pool_s[...] += pm
        cnt_s[...] += cm

    @pl.when(i == NBLK - 1)
    def _():
        pooled = pool_s[...] / jnp.maximum(cnt_s[...], 1.0)
        logits = jnp.dot(pooled, wc_ref[...],
                         preferred_element_type=jnp.float32,
                         precision=_PREC) + bc_ref[...]
        mx = jnp.max(logits, axis=1, keepdims=True)
        ez = jnp.exp(logits - mx)
        o_ref[...] = logits - mx - jnp.log(jnp.sum(ez, axis=1, keepdims=True))


def _tc_fin(a0, a1, hp, b, d0, d1, bt, wc, bc):
    return pl.pallas_call(
        _tc_fin_body,
        grid=(NBLK,),
        in_specs=[
            pl.BlockSpec((R, H), lambda i: (i, 0)),
            pl.BlockSpec((R, H), lambda i: (i, 0)),
            pl.BlockSpec((R, H), lambda i: (i, 0)),
            pl.BlockSpec((1, H), lambda i: (0, 0)),
            pl.BlockSpec((R, DW), lambda i: (i, 0)),
            pl.BlockSpec((R, DW), lambda i: (i, 0)),
            pl.BlockSpec((R, 1), lambda i: (i, 0)),
            pl.BlockSpec((H, C), lambda i: (0, 0)),
            pl.BlockSpec((1, C), lambda i: (0, 0)),
        ],
        out_specs=pl.BlockSpec((G, C), lambda i: (0, 0)),
        out_shape=jax.ShapeDtypeStruct((G, C), jnp.float32),
        scratch_shapes=[
            pltpu.VMEM((G, H), jnp.float32),
            pltpu.VMEM((G, H), jnp.float32),
        ],
    )(a0, a1, hp, b, d0, d1, bt, wc, bc)


def _split(m):
    return m[:NPAD], m[NPAD:]


def kernel(x, edge_index, batch, W1, b1, W2, b2, W3, b3, Wc, bc):
    pad = EPAD - E
    cyc = jnp.arange(pad, dtype=jnp.int32) % (NPAD - N)
    srcp = jnp.concatenate([edge_index[0], cyc])
    dstp = jnp.concatenate([edge_index[1], N + cyc]).reshape(NW, NCH, K)

    mdeg = _deg_kernel(dstp)
    d0 = mdeg[:NPAD, :DW]
    d1 = mdeg[NPAD:, :DW]

    h1p = _tc1(x, W1, d0, d1)
    m1a, m1b = _split(_msg_kernel(h1p, srcp, dstp))
    h2p = _tc_mid(m1a, m1b, h1p, b1.reshape(1, H), W2, d0, d1)
    m2a, m2b = _split(_msg_kernel(h2p, srcp, dstp))
    h3p = _tc_mid(m2a, m2b, h2p, b2.reshape(1, H), W3, d0, d1)
    m3a, m3b = _split(_msg_kernel(h3p, srcp, dstp))
    return _tc_fin(m3a, m3b, h3p, b3.reshape(1, H), d0, d1,
                   batch.reshape(N, 1), Wc, bc.reshape(1, C))


# speedup vs baseline: 22.9411x; 1.1150x over previous
import functools

import jax
import jax.numpy as jnp
from jax import lax
from jax.experimental import pallas as pl
from jax.experimental.pallas import tpu as pltpu
from jax.experimental.pallas import tpu_sc as plsc

N = 10000
E = 320000
H = 128
C = 10
G = 64

NC, NS, L = 2, 16, 16
NW = NC * NS
K = 128
NCH = 80
EPT = NCH * K
EPAD = NW * EPT
NPAD = NS * 632
ZR = NPAD // NS
DW = 16

_mesh = plsc.VectorSubcoreMesh(
    core_axis_name="c", subcore_axis_name="s", num_cores=NC, num_subcores=NS)


@functools.partial(
    pl.kernel,
    out_type=jax.ShapeDtypeStruct((NC * NPAD, H), jnp.float32),
    mesh=_mesh,
    scratch_types=[
        pltpu.VMEM((EPT // 2,), jnp.int32),
        pltpu.VMEM((NCH // 2, K), jnp.int32),
        pltpu.VMEM((K, H), jnp.float32),
        pltpu.VMEM((K, H), jnp.float32),
        pltpu.VMEM_SHARED((NPAD, H), jnp.float32),
        pltpu.SemaphoreType.DMA,
        pltpu.SemaphoreType.DMA,
    ],
)
def _msg_kernel(h_hbm, src_hbm, dst_hbm, out_hbm, sidx_v, didx_v, rows0,
                rows1, acc_sh, sem0, sem1):
    c = lax.axis_index("c")
    s = lax.axis_index("s")
    w = c * NS + s

    def _fill_zero(i, carry):
        for u in range(H // L):
            rows0[i, pl.ds(u * L, L)] = jnp.zeros((L,), jnp.float32)
        return carry

    lax.fori_loop(0, K, _fill_zero, 0)
    base = s * ZR
    off = 0
    while off < ZR:
        n = min(K, ZR - off)
        pltpu.sync_copy(rows0.at[pl.ds(0, n)],
                        acc_sh.at[pl.ds(base + off, n)])
        off += n

    plsc.subcore_barrier()

    NCH2 = NCH // 2
    for half in range(2):
        pltpu.sync_copy(
            src_hbm.at[pl.ds(w * EPT + half * (EPT // 2), EPT // 2)], sidx_v)
        pltpu.sync_copy(dst_hbm.at[w, pl.ds(half * NCH2, NCH2)], didx_v)
        pltpu.async_copy(h_hbm.at[sidx_v.at[pl.ds(0, K)]], rows0, sem0)
        pltpu.async_copy(h_hbm.at[sidx_v.at[pl.ds(K, K)]], rows1, sem1)

        def _pair(i, carry):
            j0 = 2 * i
            pltpu.make_async_copy(
                h_hbm.at[sidx_v.at[pl.ds(j0 * K, K)]], rows0, sem0).wait()
            pltpu.sync_copy(rows0, acc_sh.at[didx_v.at[j0]], add=True)
            nxt0 = jnp.minimum(j0 + 2, NCH2 - 1) * K
            pltpu.async_copy(h_hbm.at[sidx_v.at[pl.ds(nxt0, K)]], rows0, sem0)
            pltpu.make_async_copy(
                h_hbm.at[sidx_v.at[pl.ds(j0 * K + K, K)]], rows1, sem1).wait()
            pltpu.sync_copy(rows1, acc_sh.at[didx_v.at[j0 + 1]], add=True)
            nxt1 = jnp.minimum(j0 + 3, NCH2 - 1) * K
            pltpu.async_copy(h_hbm.at[sidx_v.at[pl.ds(nxt1, K)]], rows1, sem1)
            return carry

        lax.fori_loop(0, NCH2 // 2, _pair, 0)
        pltpu.make_async_copy(
            h_hbm.at[sidx_v.at[pl.ds(0, K)]], rows0, sem0).wait()
        pltpu.make_async_copy(
            h_hbm.at[sidx_v.at[pl.ds(0, K)]], rows1, sem1).wait()
    plsc.subcore_barrier()
    pltpu.sync_copy(acc_sh.at[pl.ds(s * ZR, ZR)],
                    out_hbm.at[pl.ds(c * NPAD + s * ZR, ZR)])


@functools.partial(
    pl.kernel,
    out_type=jax.ShapeDtypeStruct((NC * NPAD, H), jnp.float32),
    mesh=_mesh,
    scratch_types=[
        pltpu.VMEM((NCH, K), jnp.int32),
        pltpu.VMEM((K, H), jnp.float32),
        pltpu.VMEM_SHARED((NPAD, H), jnp.float32),
    ],
)
def _deg_kernel(dst_hbm, out_hbm, didx_v, ones_v, acc_sh):
    c = lax.axis_index("c")
    s = lax.axis_index("s")
    w = c * NS + s

    def _fill(i, carry):
        for u in range(H // L):
            ones_v[i, pl.ds(u * L, L)] = jnp.zeros((L,), jnp.float32)
        return carry

    lax.fori_loop(0, K, _fill, 0)
    base = s * ZR
    off = 0
    while off < ZR:
        n = min(K, ZR - off)
        pltpu.sync_copy(ones_v.at[pl.ds(0, n)],
                        acc_sh.at[pl.ds(base + off, n)])
        off += n

    def _refill(i, carry):
        for u in range(H // L):
            ones_v[i, pl.ds(u * L, L)] = jnp.ones((L,), jnp.float32)
        return carry

    lax.fori_loop(0, K, _refill, 0)
    pltpu.sync_copy(dst_hbm.at[w], didx_v)
    plsc.subcore_barrier()

    def _chunk(j, carry):
        pltpu.sync_copy(ones_v, acc_sh.at[didx_v.at[j]], add=True)
        return carry

    lax.fori_loop(0, NCH, _chunk, 0)
    plsc.subcore_barrier()
    pltpu.sync_copy(acc_sh.at[pl.ds(s * ZR, ZR)],
                    out_hbm.at[pl.ds(c * NPAD + s * ZR, ZR)])


R = 2000
NBLK = N // R
_PREC = lax.Precision.HIGHEST


def _dinv_of(d0_ref, d1_ref):
    return lax.rsqrt(d0_ref[:, 0:1] + d1_ref[:, 0:1] + 1.0)


def _tc1_body(x_ref, w_ref, d0_ref, d1_ref, o_ref):
    dinv = _dinv_of(d0_ref, d1_ref)
    o_ref[...] = dinv * jnp.dot(x_ref[...], w_ref[...],
                                preferred_element_type=jnp.float32,
                                precision=_PREC)


def _tc1(x, w, d0, d1):
    return pl.pallas_call(
        _tc1_body,
        grid=(NBLK,),
        in_specs=[
            pl.BlockSpec((R, H), lambda i: (i, 0)),
            pl.BlockSpec((H, H), lambda i: (0, 0)),
            pl.BlockSpec((R, DW), lambda i: (i, 0)),
            pl.BlockSpec((R, DW), lambda i: (i, 0)),
        ],
        out_specs=pl.BlockSpec((R, H), lambda i: (i, 0)),
        out_shape=jax.ShapeDtypeStruct((N, H), jnp.float32),
    )(x, w, d0, d1)


def _tc_mid_body(a0_ref, a1_ref, hp_ref, b_ref, w_ref, d0_ref, d1_ref, o_ref):
    dinv = _dinv_of(d0_ref, d1_ref)
    xn = jnp.maximum(
        dinv * (a0_ref[...] + a1_ref[...] + hp_ref[...]) + b_ref[...], 0.0)
    o_ref[...] = dinv * jnp.dot(xn, w_ref[...],
                                preferred_element_type=jnp.float32,
                                precision=_PREC)


def _tc_mid(a0, a1, hp, b, w, d0, d1):
    return pl.pallas_call(
        _tc_mid_body,
        grid=(NBLK,),
        in_specs=[
            pl.BlockSpec((R, H), lambda i: (i, 0)),
            pl.BlockSpec((R, H), lambda i: (i, 0)),
            pl.BlockSpec((R, H), lambda i: (i, 0)),
            pl.BlockSpec((1, H), lambda i: (0, 0)),
            pl.BlockSpec((H, H), lambda i: (0, 0)),
            pl.BlockSpec((R, DW), lambda i: (i, 0)),
            pl.BlockSpec((R, DW), lambda i: (i, 0)),
        ],
        out_specs=pl.BlockSpec((R, H), lambda i: (i, 0)),
        out_shape=jax.ShapeDtypeStruct((N, H), jnp.float32),
    )(a0, a1, hp, b, w, d0, d1)


def _tc_fin_body(a0_ref, a1_ref, hp_ref, b_ref, d0_ref, d1_ref, bt_ref,
                 wc_ref, bc_ref, o_ref, pool_s, cnt_s):
    i = pl.program_id(0)
    dinv = _dinv_of(d0_ref, d1_ref)
    x3 = jnp.maximum(
        dinv * (a0_ref[...] + a1_ref[...] + hp_ref[...]) + b_ref[...], 0.0)
    gid = lax.broadcasted_iota(jnp.int32, (R, G), 1)
    m = (gid == bt_ref[...]).astype(jnp.float32)
    dn = (((0,), (0,)), ((), ()))
    pm = lax.dot_general(m, x3, dimension_numbers=dn,
                         preferred_element_type=jnp.float32, precision=_PREC)
    cm = lax.dot_general(m, jnp.ones((R, H), jnp.float32),
                         dimension_numbers=dn,
                         preferred_element_type=jnp.float32, precision=_PREC)

    @pl.when(i == 0)
    def _():
        pool_s[...] = pm
        cnt_s[...] = cm

    @pl.when(i > 0)
    def _():
        pool_s[...] += pm
        cnt_s[...] += cm

    @pl.when(i == NBLK - 1)
    def _():
        pooled = pool_s[...] / jnp.maximum(cnt_s[...], 1.0)
        logits = jnp.dot(pooled, wc_ref[...],
                         preferred_element_type=jnp.float32,
                         precision=_PREC) + bc_ref[...]
        mx = jnp.max(logits, axis=1, keepdims=True)
        ez = jnp.exp(logits - mx)
        o_ref[...] = logits - mx - jnp.log(jnp.sum(ez, axis=1, keepdims=True))


def _tc_fin(a0, a1, hp, b, d0, d1, bt, wc, bc):
    return pl.pallas_call(
        _tc_fin_body,
        grid=(NBLK,),
        in_specs=[
            pl.BlockSpec((R, H), lambda i: (i, 0)),
            pl.BlockSpec((R, H), lambda i: (i, 0)),
            pl.BlockSpec((R, H), lambda i: (i, 0)),
            pl.BlockSpec((1, H), lambda i: (0, 0)),
            pl.BlockSpec((R, DW), lambda i: (i, 0)),
            pl.BlockSpec((R, DW), lambda i: (i, 0)),
            pl.BlockSpec((R, 1), lambda i: (i, 0)),
            pl.BlockSpec((H, C), lambda i: (0, 0)),
            pl.BlockSpec((1, C), lambda i: (0, 0)),
        ],
        out_specs=pl.BlockSpec((G, C), lambda i: (0, 0)),
        out_shape=jax.ShapeDtypeStruct((G, C), jnp.float32),
        scratch_shapes=[
            pltpu.VMEM((G, H), jnp.float32),
            pltpu.VMEM((G, H), jnp.float32),
        ],
    )(a0, a1, hp, b, d0, d1, bt, wc, bc)


def _split(m):
    return m[:NPAD], m[NPAD:]


def kernel(x, edge_index, batch, W1, b1, W2, b2, W3, b3, Wc, bc):
    pad = EPAD - E
    cyc = jnp.arange(pad, dtype=jnp.int32) % (NPAD - N)
    srcp = jnp.concatenate([edge_index[0], cyc])
    dstp = jnp.concatenate([edge_index[1], N + cyc]).reshape(NW, NCH, K)

    mdeg = _deg_kernel(dstp)
    d0 = mdeg[:NPAD, :DW]
    d1 = mdeg[NPAD:, :DW]

    h1p = _tc1(x, W1, d0, d1)
    m1a, m1b = _split(_msg_kernel(h1p, srcp, dstp))
    h2p = _tc_mid(m1a, m1b, h1p, b1.reshape(1, H), W2, d0, d1)
    m2a, m2b = _split(_msg_kernel(h2p, srcp, dstp))
    h3p = _tc_mid(m2a, m2b, h2p, b2.reshape(1, H), W3, d0, d1)
    m3a, m3b = _split(_msg_kernel(h3p, srcp, dstp))
    return _tc_fin(m3a, m3b, h3p, b3.reshape(1, H), d0, d1,
                   batch.reshape(N, 1), Wc, bc.reshape(1, C))


# speedup vs baseline: 25.2161x; 1.0992x over previous
import functools

import jax
import jax.numpy as jnp
from jax import lax
from jax.experimental import pallas as pl
from jax.experimental.pallas import tpu as pltpu
from jax.experimental.pallas import tpu_sc as plsc

N = 10000
E = 320000
H = 128
C = 10
G = 64

NC, NS, L = 2, 16, 16
NW = NC * NS
K = 128
NCH = 80
EPT = NCH * K
EPAD = NW * EPT
NPAD = NS * 632
ZR = NPAD // NS
DW = 16

_mesh = plsc.VectorSubcoreMesh(
    core_axis_name="c", subcore_axis_name="s", num_cores=NC, num_subcores=NS)


KM = 64
NB = 4
NCHM = EPT // KM
NCHM2 = NCHM // 2


@functools.partial(
    pl.kernel,
    out_type=jax.ShapeDtypeStruct((NC * NPAD, H), jnp.float32),
    mesh=_mesh,
    scratch_types=[
        pltpu.VMEM((EPT // 2,), jnp.int32),
        pltpu.VMEM((NCHM2, KM), jnp.int32),
        pltpu.VMEM((KM, H), jnp.float32),
        pltpu.VMEM((KM, H), jnp.float32),
        pltpu.VMEM((KM, H), jnp.float32),
        pltpu.VMEM((KM, H), jnp.float32),
        pltpu.VMEM_SHARED((NPAD, H), jnp.float32),
        pltpu.SemaphoreType.DMA,
        pltpu.SemaphoreType.DMA,
        pltpu.SemaphoreType.DMA,
        pltpu.SemaphoreType.DMA,
    ],
)
def _msg_kernel(h_hbm, src_hbm, dst_hbm, out_hbm, sidx_v, didx_v, rows0,
                rows1, rows2, rows3, acc_sh, sem0, sem1, sem2, sem3):
    c = lax.axis_index("c")
    s = lax.axis_index("s")
    w = c * NS + s
    rows = [rows0, rows1, rows2, rows3]
    sems = [sem0, sem1, sem2, sem3]

    def _fill_zero(i, carry):
        for u in range(H // L):
            rows0[i, pl.ds(u * L, L)] = jnp.zeros((L,), jnp.float32)
        return carry

    lax.fori_loop(0, KM, _fill_zero, 0)
    base = s * ZR
    off = 0
    while off < ZR:
        n = min(KM, ZR - off)
        pltpu.sync_copy(rows0.at[pl.ds(0, n)],
                        acc_sh.at[pl.ds(base + off, n)])
        off += n

    plsc.subcore_barrier()

    for half in range(2):
        pltpu.sync_copy(
            src_hbm.at[pl.ds(w * EPT + half * (EPT // 2), EPT // 2)], sidx_v)
        pltpu.sync_copy(dst_hbm.at[w, pl.ds(half * NCHM2, NCHM2)], didx_v)
        for b in range(NB - 1):
            pltpu.async_copy(
                h_hbm.at[sidx_v.at[pl.ds(b * KM, KM)]], rows[b], sems[b])

        def _ring(i, carry):
            j0 = NB * i
            for b in range(NB):
                j = j0 + b
                pltpu.make_async_copy(
                    h_hbm.at[sidx_v.at[pl.ds(j * KM, KM)]],
                    rows[b], sems[b]).wait()
                nxt = jnp.minimum(j + NB - 1, NCHM2 - 1) * KM
                bn = (b + NB - 1) % NB
                pltpu.async_copy(
                    h_hbm.at[sidx_v.at[pl.ds(nxt, KM)]], rows[bn], sems[bn])
                pltpu.sync_copy(rows[b], acc_sh.at[didx_v.at[j]], add=True)
            return carry

        lax.fori_loop(0, NCHM2 // NB, _ring, 0)
        for b in range(NB - 1):
            pltpu.make_async_copy(
                h_hbm.at[sidx_v.at[pl.ds(0, KM)]], rows[b], sems[b]).wait()
    plsc.subcore_barrier()
    pltpu.sync_copy(acc_sh.at[pl.ds(s * ZR, ZR)],
                    out_hbm.at[pl.ds(c * NPAD + s * ZR, ZR)])


@functools.partial(
    pl.kernel,
    out_type=jax.ShapeDtypeStruct((NC * NPAD, H), jnp.float32),
    mesh=_mesh,
    scratch_types=[
        pltpu.VMEM((NCH, K), jnp.int32),
        pltpu.VMEM((K, H), jnp.float32),
        pltpu.VMEM_SHARED((NPAD, H), jnp.float32),
    ],
)
def _deg_kernel(dst_hbm, out_hbm, didx_v, ones_v, acc_sh):
    c = lax.axis_index("c")
    s = lax.axis_index("s")
    w = c * NS + s

    def _fill(i, carry):
        for u in range(H // L):
            ones_v[i, pl.ds(u * L, L)] = jnp.zeros((L,), jnp.float32)
        return carry

    lax.fori_loop(0, K, _fill, 0)
    base = s * ZR
    off = 0
    while off < ZR:
        n = min(K, ZR - off)
        pltpu.sync_copy(ones_v.at[pl.ds(0, n)],
                        acc_sh.at[pl.ds(base + off, n)])
        off += n

    def _refill(i, carry):
        for u in range(H // L):
            ones_v[i, pl.ds(u * L, L)] = jnp.ones((L,), jnp.float32)
        return carry

    lax.fori_loop(0, K, _refill, 0)
    pltpu.sync_copy(dst_hbm.at[w], didx_v)
    plsc.subcore_barrier()

    def _chunk(j, carry):
        pltpu.sync_copy(ones_v, acc_sh.at[didx_v.at[j]], add=True)
        return carry

    lax.fori_loop(0, NCH, _chunk, 0)
    plsc.subcore_barrier()
    pltpu.sync_copy(acc_sh.at[pl.ds(s * ZR, ZR)],
                    out_hbm.at[pl.ds(c * NPAD + s * ZR, ZR)])


R = 2000
NBLK = N // R
_PREC = lax.Precision.HIGHEST


def _dinv_of(d0_ref, d1_ref):
    return lax.rsqrt(d0_ref[:, 0:1] + d1_ref[:, 0:1] + 1.0)


def _tc1_body(x_ref, w_ref, d0_ref, d1_ref, o_ref):
    dinv = _dinv_of(d0_ref, d1_ref)
    o_ref[...] = dinv * jnp.dot(x_ref[...], w_ref[...],
                                preferred_element_type=jnp.float32,
                                precision=_PREC)


def _tc1(x, w, d0, d1):
    return pl.pallas_call(
        _tc1_body,
        grid=(NBLK,),
        in_specs=[
            pl.BlockSpec((R, H), lambda i: (i, 0)),
            pl.BlockSpec((H, H), lambda i: (0, 0)),
            pl.BlockSpec((R, DW), lambda i: (i, 0)),
            pl.BlockSpec((R, DW), lambda i: (i, 0)),
        ],
        out_specs=pl.BlockSpec((R, H), lambda i: (i, 0)),
        out_shape=jax.ShapeDtypeStruct((N, H), jnp.float32),
    )(x, w, d0, d1)


def _tc_mid_body(a0_ref, a1_ref, hp_ref, b_ref, w_ref, d0_ref, d1_ref, o_ref):
    dinv = _dinv_of(d0_ref, d1_ref)
    xn = jnp.maximum(
        dinv * (a0_ref[...] + a1_ref[...] + hp_ref[...]) + b_ref[...], 0.0)
    o_ref[...] = dinv * jnp.dot(xn, w_ref[...],
                                preferred_element_type=jnp.float32,
                                precision=_PREC)


def _tc_mid(a0, a1, hp, b, w, d0, d1):
    return pl.pallas_call(
        _tc_mid_body,
        grid=(NBLK,),
        in_specs=[
            pl.BlockSpec((R, H), lambda i: (i, 0)),
            pl.BlockSpec((R, H), lambda i: (i, 0)),
            pl.BlockSpec((R, H), lambda i: (i, 0)),
            pl.BlockSpec((1, H), lambda i: (0, 0)),
            pl.BlockSpec((H, H), lambda i: (0, 0)),
            pl.BlockSpec((R, DW), lambda i: (i, 0)),
            pl.BlockSpec((R, DW), lambda i: (i, 0)),
        ],
        out_specs=pl.BlockSpec((R, H), lambda i: (i, 0)),
        out_shape=jax.ShapeDtypeStruct((N, H), jnp.float32),
    )(a0, a1, hp, b, w, d0, d1)


def _tc_fin_body(a0_ref, a1_ref, hp_ref, b_ref, d0_ref, d1_ref, bt_ref,
                 wc_ref, bc_ref, o_ref, pool_s, cnt_s):
    i = pl.program_id(0)
    dinv = _dinv_of(d0_ref, d1_ref)
    x3 = jnp.maximum(
        dinv * (a0_ref[...] + a1_ref[...] + hp_ref[...]) + b_ref[...], 0.0)
    gid = lax.broadcasted_iota(jnp.int32, (R, G), 1)
    m = (gid == bt_ref[...]).astype(jnp.float32)
    dn = (((0,), (0,)), ((), ()))
    pm = lax.dot_general(m, x3, dimension_numbers=dn,
                         preferred_element_type=jnp.float32, precision=_PREC)
    cm = lax.dot_general(m, jnp.ones((R, H), jnp.float32),
                         dimension_numbers=dn,
                         preferred_element_type=jnp.float32, precision=_PREC)

    @pl.when(i == 0)
    def _():
        pool_s[...] = pm
        cnt_s[...] = cm

    @pl.when(i > 0)
    def _():
        pool_s[...] += pm
        cnt_s[...] += cm

    @pl.when(i == NBLK - 1)
    def _():
        pooled = pool_s[...] / jnp.maximum(cnt_s[...], 1.0)
        logits = jnp.dot(pooled, wc_ref[...],
                         preferred_element_type=jnp.float32,
                         precision=_PREC) + bc_ref[...]
        mx = jnp.max(logits, axis=1, keepdims=True)
        ez = jnp.exp(logits - mx)
        o_ref[...] = logits - mx - jnp.log(jnp.sum(ez, axis=1, keepdims=True))


def _tc_fin(a0, a1, hp, b, d0, d1, bt, wc, bc):
    return pl.pallas_call(
        _tc_fin_body,
        grid=(NBLK,),
        in_specs=[
            pl.BlockSpec((R, H), lambda i: (i, 0)),
            pl.BlockSpec((R, H), lambda i: (i, 0)),
            pl.BlockSpec((R, H), lambda i: (i, 0)),
            pl.BlockSpec((1, H), lambda i: (0, 0)),
            pl.BlockSpec((R, DW), lambda i: (i, 0)),
            pl.BlockSpec((R, DW), lambda i: (i, 0)),
            pl.BlockSpec((R, 1), lambda i: (i, 0)),
            pl.BlockSpec((H, C), lambda i: (0, 0)),
            pl.BlockSpec((1, C), lambda i: (0, 0)),
        ],
        out_specs=pl.BlockSpec((G, C), lambda i: (0, 0)),
        out_shape=jax.ShapeDtypeStruct((G, C), jnp.float32),
        scratch_shapes=[
            pltpu.VMEM((G, H), jnp.float32),
            pltpu.VMEM((G, H), jnp.float32),
        ],
    )(a0, a1, hp, b, d0, d1, bt, wc, bc)


def _split(m):
    return m[:NPAD], m[NPAD:]


def kernel(x, edge_index, batch, W1, b1, W2, b2, W3, b3, Wc, bc):
    pad = EPAD - E
    cyc = jnp.arange(pad, dtype=jnp.int32) % (NPAD - N)
    srcp = jnp.concatenate([edge_index[0], cyc])
    dstp = jnp.concatenate([edge_index[1], N + cyc])
    dst_deg = dstp.reshape(NW, NCH, K)
    dst_msg = dstp.reshape(NW, NCHM, KM)

    mdeg = _deg_kernel(dst_deg)
    d0 = mdeg[:NPAD, :DW]
    d1 = mdeg[NPAD:, :DW]

    h1p = _tc1(x, W1, d0, d1)
    m1a, m1b = _split(_msg_kernel(h1p, srcp, dst_msg))
    h2p = _tc_mid(m1a, m1b, h1p, b1.reshape(1, H), W2, d0, d1)
    m2a, m2b = _split(_msg_kernel(h2p, srcp, dst_msg))
    h3p = _tc_mid(m2a, m2b, h2p, b2.reshape(1, H), W3, d0, d1)
    m3a, m3b = _split(_msg_kernel(h3p, srcp, dst_msg))
    return _tc_fin(m3a, m3b, h3p, b3.reshape(1, H), d0, d1,
                   batch.reshape(N, 1), Wc, bc.reshape(1, C))


# speedup vs baseline: 25.2381x; 1.0009x over previous
import functools

import jax
import jax.numpy as jnp
from jax import lax
from jax.experimental import pallas as pl
from jax.experimental.pallas import tpu as pltpu
from jax.experimental.pallas import tpu_sc as plsc

N = 10000
E = 320000
H = 128
C = 10
G = 64

NC, NS, L = 2, 16, 16
NW = NC * NS
K = 128
NCH = 80
EPT = NCH * K
EPAD = NW * EPT
NPAD = NS * 632
ZR = NPAD // NS
DW = 16

_mesh = plsc.VectorSubcoreMesh(
    core_axis_name="c", subcore_axis_name="s", num_cores=NC, num_subcores=NS)


KM = 64
NB = 4
NCHM = EPT // KM
NCHM2 = NCHM // 2


@functools.partial(
    pl.kernel,
    out_type=jax.ShapeDtypeStruct((NC * NPAD, H), jnp.float32),
    mesh=_mesh,
    scratch_types=[
        pltpu.VMEM((EPT // 2,), jnp.int32),
        pltpu.VMEM((NCHM2, KM), jnp.int32),
        pltpu.VMEM((KM, H), jnp.float32),
        pltpu.VMEM((KM, H), jnp.float32),
        pltpu.VMEM((KM, H), jnp.float32),
        pltpu.VMEM((KM, H), jnp.float32),
        pltpu.VMEM_SHARED((NPAD, H), jnp.float32),
        pltpu.SemaphoreType.DMA,
        pltpu.SemaphoreType.DMA,
        pltpu.SemaphoreType.DMA,
        pltpu.SemaphoreType.DMA,
    ],
)
def _msg_kernel(h_hbm, src_hbm, dst_hbm, out_hbm, sidx_v, didx_v, rows0,
                rows1, rows2, rows3, acc_sh, sem0, sem1, sem2, sem3):
    c = lax.axis_index("c")
    s = lax.axis_index("s")
    w = c * NS + s
    rows = [rows0, rows1, rows2, rows3]
    sems = [sem0, sem1, sem2, sem3]

    def _fill_zero(i, carry):
        for u in range(H // L):
            rows0[i, pl.ds(u * L, L)] = jnp.zeros((L,), jnp.float32)
        return carry

    lax.fori_loop(0, KM, _fill_zero, 0)
    base = s * ZR
    off = 0
    while off < ZR:
        n = min(KM, ZR - off)
        pltpu.sync_copy(rows0.at[pl.ds(0, n)],
                        acc_sh.at[pl.ds(base + off, n)])
        off += n

    plsc.subcore_barrier()

    for half in range(2):
        pltpu.sync_copy(
            src_hbm.at[pl.ds(w * EPT + half * (EPT // 2), EPT // 2)], sidx_v)
        pltpu.sync_copy(dst_hbm.at[w, pl.ds(half * NCHM2, NCHM2)], didx_v)
        for b in range(NB - 1):
            pltpu.async_copy(
                h_hbm.at[sidx_v.at[pl.ds(b * KM, KM)]], rows[b], sems[b])

        def _ring(i, carry):
            j0 = NB * i
            for b in range(NB):
                j = j0 + b
                pltpu.make_async_copy(
                    h_hbm.at[sidx_v.at[pl.ds(j * KM, KM)]],
                    rows[b], sems[b]).wait()
                nxt = jnp.minimum(j + NB - 1, NCHM2 - 1) * KM
                bn = (b + NB - 1) % NB
                pltpu.async_copy(
                    h_hbm.at[sidx_v.at[pl.ds(nxt, KM)]], rows[bn], sems[bn])
                pltpu.sync_copy(rows[b], acc_sh.at[didx_v.at[j]], add=True)
            return carry

        lax.fori_loop(0, NCHM2 // NB, _ring, 0)
        for b in range(NB - 1):
            pltpu.make_async_copy(
                h_hbm.at[sidx_v.at[pl.ds(0, KM)]], rows[b], sems[b]).wait()
    plsc.subcore_barrier()
    pltpu.sync_copy(acc_sh.at[pl.ds(s * ZR, ZR)],
                    out_hbm.at[pl.ds(c * NPAD + s * ZR, ZR)])


@functools.partial(
    pl.kernel,
    out_type=jax.ShapeDtypeStruct((NC * NPAD, H), jnp.float32),
    mesh=_mesh,
    scratch_types=[
        pltpu.VMEM((NCH, K), jnp.int32),
        pltpu.VMEM((K, H), jnp.float32),
        pltpu.VMEM_SHARED((NPAD, H), jnp.float32),
        pltpu.SemaphoreType.DMA,
        pltpu.SemaphoreType.DMA,
    ],
)
def _deg_kernel(dst_hbm, out_hbm, didx_v, ones_v, acc_sh, sem0, sem1):
    c = lax.axis_index("c")
    s = lax.axis_index("s")
    w = c * NS + s

    def _fill(i, carry):
        for u in range(H // L):
            ones_v[i, pl.ds(u * L, L)] = jnp.zeros((L,), jnp.float32)
        return carry

    lax.fori_loop(0, K, _fill, 0)
    base = s * ZR
    off = 0
    while off < ZR:
        n = min(K, ZR - off)
        pltpu.sync_copy(ones_v.at[pl.ds(0, n)],
                        acc_sh.at[pl.ds(base + off, n)])
        off += n

    def _refill(i, carry):
        for u in range(H // L):
            ones_v[i, pl.ds(u * L, L)] = jnp.ones((L,), jnp.float32)
        return carry

    lax.fori_loop(0, K, _refill, 0)
    pltpu.sync_copy(dst_hbm.at[w], didx_v)
    plsc.subcore_barrier()

    def _chunk(i, carry):
        j0 = 2 * i
        pltpu.async_copy(ones_v, acc_sh.at[didx_v.at[j0]], sem0, add=True)
        pltpu.async_copy(ones_v, acc_sh.at[didx_v.at[j0 + 1]], sem1, add=True)
        pltpu.make_async_copy(
            ones_v, acc_sh.at[didx_v.at[j0]], sem0).wait()
        pltpu.make_async_copy(
            ones_v, acc_sh.at[didx_v.at[j0 + 1]], sem1).wait()
        return carry

    lax.fori_loop(0, NCH // 2, _chunk, 0)
    plsc.subcore_barrier()
    pltpu.sync_copy(acc_sh.at[pl.ds(s * ZR, ZR)],
                    out_hbm.at[pl.ds(c * NPAD + s * ZR, ZR)])


R = 2000
NBLK = N // R
_PREC = lax.Precision.HIGHEST


def _dinv_of(d0_ref, d1_ref):
    return lax.rsqrt(d0_ref[:, 0:1] + d1_ref[:, 0:1] + 1.0)


def _tc1_body(x_ref, w_ref, d0_ref, d1_ref, o_ref):
    dinv = _dinv_of(d0_ref, d1_ref)
    o_ref[...] = dinv * jnp.dot(x_ref[...], w_ref[...],
                                preferred_element_type=jnp.float32,
                                precision=_PREC)


def _tc1(x, w, d0, d1):
    return pl.pallas_call(
        _tc1_body,
        grid=(NBLK,),
        in_specs=[
            pl.BlockSpec((R, H), lambda i: (i, 0)),
            pl.BlockSpec((H, H), lambda i: (0, 0)),
            pl.BlockSpec((R, DW), lambda i: (i, 0)),
            pl.BlockSpec((R, DW), lambda i: (i, 0)),
        ],
        out_specs=pl.BlockSpec((R, H), lambda i: (i, 0)),
        out_shape=jax.ShapeDtypeStruct((N, H), jnp.float32),
    )(x, w, d0, d1)


def _tc_mid_body(a0_ref, a1_ref, hp_ref, b_ref, w_ref, d0_ref, d1_ref, o_ref):
    dinv = _dinv_of(d0_ref, d1_ref)
    xn = jnp.maximum(
        dinv * (a0_ref[...] + a1_ref[...] + hp_ref[...]) + b_ref[...], 0.0)
    o_ref[...] = dinv * jnp.dot(xn, w_ref[...],
                                preferred_element_type=jnp.float32,
                                precision=_PREC)


def _tc_mid(a0, a1, hp, b, w, d0, d1):
    return pl.pallas_call(
        _tc_mid_body,
        grid=(NBLK,),
        in_specs=[
            pl.BlockSpec((R, H), lambda i: (i, 0)),
            pl.BlockSpec((R, H), lambda i: (i, 0)),
            pl.BlockSpec((R, H), lambda i: (i, 0)),
            pl.BlockSpec((1, H), lambda i: (0, 0)),
            pl.BlockSpec((H, H), lambda i: (0, 0)),
            pl.BlockSpec((R, DW), lambda i: (i, 0)),
            pl.BlockSpec((R, DW), lambda i: (i, 0)),
        ],
        out_specs=pl.BlockSpec((R, H), lambda i: (i, 0)),
        out_shape=jax.ShapeDtypeStruct((N, H), jnp.float32),
    )(a0, a1, hp, b, w, d0, d1)


def _tc_fin_body(a0_ref, a1_ref, hp_ref, b_ref, d0_ref, d1_ref, bt_ref,
                 wc_ref, bc_ref, o_ref, pool_s, cnt_s):
    i = pl.program_id(0)
    dinv = _dinv_of(d0_ref, d1_ref)
    x3 = jnp.maximum(
        dinv * (a0_ref[...] + a1_ref[...] + hp_ref[...]) + b_ref[...], 0.0)
    gid = lax.broadcasted_iota(jnp.int32, (R, G), 1)
    m = (gid == bt_ref[...]).astype(jnp.float32)
    dn = (((0,), (0,)), ((), ()))
    pm = lax.dot_general(m, x3, dimension_numbers=dn,
                         preferred_element_type=jnp.float32, precision=_PREC)
    cm = lax.dot_general(m, jnp.ones((R, H), jnp.float32),
                         dimension_numbers=dn,
                         preferred_element_type=jnp.float32, precision=_PREC)

    @pl.when(i == 0)
    def _():
        pool_s[...] = pm
        cnt_s[...] = cm

    @pl.when(i > 0)
    def _():
        pool_s[...] += pm
        cnt_s[...] += cm

    @pl.when(i == NBLK - 1)
    def _():
        pooled = pool_s[...] / jnp.maximum(cnt_s[...], 1.0)
        logits = jnp.dot(pooled, wc_ref[...],
                         preferred_element_type=jnp.float32,
                         precision=_PREC) + bc_ref[...]
        mx = jnp.max(logits, axis=1, keepdims=True)
        ez = jnp.exp(logits - mx)
        o_ref[...] = logits - mx - jnp.log(jnp.sum(ez, axis=1, keepdims=True))


def _tc_fin(a0, a1, hp, b, d0, d1, bt, wc, bc):
    return pl.pallas_call(
        _tc_fin_body,
        grid=(NBLK,),
        in_specs=[
            pl.BlockSpec((R, H), lambda i: (i, 0)),
            pl.BlockSpec((R, H), lambda i: (i, 0)),
            pl.BlockSpec((R, H), lambda i: (i, 0)),
            pl.BlockSpec((1, H), lambda i: (0, 0)),
            pl.BlockSpec((R, DW), lambda i: (i, 0)),
            pl.BlockSpec((R, DW), lambda i: (i, 0)),
            pl.BlockSpec((R, 1), lambda i: (i, 0)),
            pl.BlockSpec((H, C), lambda i: (0, 0)),
            pl.BlockSpec((1, C), lambda i: (0, 0)),
        ],
        out_specs=pl.BlockSpec((G, C), lambda i: (0, 0)),
        out_shape=jax.ShapeDtypeStruct((G, C), jnp.float32),
        scratch_shapes=[
            pltpu.VMEM((G, H), jnp.float32),
            pltpu.VMEM((G, H), jnp.float32),
        ],
    )(a0, a1, hp, b, d0, d1, bt, wc, bc)


def _split(m):
    return m[:NPAD], m[NPAD:]


def kernel(x, edge_index, batch, W1, b1, W2, b2, W3, b3, Wc, bc):
    pad = EPAD - E
    cyc = jnp.arange(pad, dtype=jnp.int32) % (NPAD - N)
    srcp = jnp.concatenate([edge_index[0], cyc])
    dstp = jnp.concatenate([edge_index[1], N + cyc])
    dst_deg = dstp.reshape(NW, NCH, K)
    dst_msg = dstp.reshape(NW, NCHM, KM)

    mdeg = _deg_kernel(dst_deg)
    d0 = mdeg[:NPAD, :DW]
    d1 = mdeg[NPAD:, :DW]

    h1p = _tc1(x, W1, d0, d1)
    m1a, m1b = _split(_msg_kernel(h1p, srcp, dst_msg))
    h2p = _tc_mid(m1a, m1b, h1p, b1.reshape(1, H), W2, d0, d1)
    m2a, m2b = _split(_msg_kernel(h2p, srcp, dst_msg))
    h3p = _tc_mid(m2a, m2b, h2p, b2.reshape(1, H), W3, d0, d1)
    m3a, m3b = _split(_msg_kernel(h3p, srcp, dst_msg))
    return _tc_fin(m3a, m3b, h3p, b3.reshape(1, H), d0, d1,
                   batch.reshape(N, 1), Wc, bc.reshape(1, C))
